# Initial kernel scaffold; baseline (speedup 1.0000x reference)
import jax
import jax.numpy as jnp
from jax.experimental import pallas as pl


def kernel(y_1, y_2, t, epoch):
    raise NotImplementedError("write your pallas kernel here")


# speedup vs baseline: 1.2122x; 1.2122x over previous
import jax
import jax.numpy as jnp
from jax import lax
from jax.experimental import pallas as pl
from jax.experimental.pallas import tpu as pltpu

_B = 16384
_C = 1000
_CP = 1024
_RB = 512
_EPOCHS = 100
_DECAY_W = 1.0
_CO_LAMBDA = 0.1
_FMIN = -3.0e38


def _rows_kernel(th_ref, ex_ref, y1_ref, y2_ref, t_ref,
                 lp_ref, ldc_ref, fc_ref, kl_ref):
    thresh = th_ref[0, 0]
    expnt = ex_ref[0, 0]
    col = lax.broadcasted_iota(jnp.int32, (_RB, _CP), 1)
    valid = col < _C
    y1 = jnp.where(valid, y1_ref[...], _FMIN)
    y2 = jnp.where(valid, y2_ref[...], _FMIN)
    m1 = jnp.max(y1, axis=1, keepdims=True)
    m2 = jnp.max(y2, axis=1, keepdims=True)
    ex1 = jnp.exp(y1 - m1)
    ex2 = jnp.exp(y2 - m2)
    e1 = jnp.sum(ex1, axis=1, keepdims=True)
    e2 = jnp.sum(ex2, axis=1, keepdims=True)
    log_e1 = jnp.log(e1)
    log_e2 = jnp.log(e2)
    lse1 = m1 + log_e1
    lse2 = m2 + log_e2
    a1 = jnp.min(jnp.where(y1 == m1, col, _C), axis=1, keepdims=True)
    a2 = jnp.min(jnp.where(y2 == m2, col, _C), axis=1, keepdims=True)
    tcol = t_ref[...]
    tm = col == tcol
    yt1 = jnp.sum(jnp.where(tm, y1, 0.0), axis=1, keepdims=True)
    yt2 = jnp.sum(jnp.where(tm, y2, 0.0), axis=1, keepdims=True)
    ydc2 = jnp.sum(jnp.where(col == a1, y2, 0.0), axis=1, keepdims=True)
    lp_ref[...] = (lse1 - yt1) + (lse2 - yt2)
    q1 = ex1 / e1
    q2 = ex2 / e2
    kl_ref[...] = jnp.sum((q1 - q2) * (y1 - y2), axis=1, keepdims=True)
    pp = (1.0 / e1) * (1.0 / e2)
    fc_ref[...] = jnp.where((a1 != tcol) & (a1 == a2) & (pp > thresh),
                            1.0, 0.0)
    aw = jnp.exp(expnt * jnp.log(pp))
    ldc_ref[...] = aw * (log_e1 + (lse2 - ydc2))


def _select_kernel(rrb_ref, lp_ref, ldc_ref, fc_ref, kl_ref, out_ref):
    loss = lp_ref[...]
    ldc = ldc_ref[...]
    fc = fc_ref[...] > 0.5
    inv_n = jnp.float32(1.0 / _B)
    mean_v = jnp.sum(loss) * inv_n
    cnt_small = jnp.sum((loss < mean_v).astype(jnp.float32))
    rr = jnp.maximum(rrb_ref[0, 0], cnt_small * inv_n)
    k = jnp.floor(rr * _B).astype(jnp.int32)
    key = lax.bitcast_convert_type(loss, jnp.int32)

    def sel_bit(i, r):
        trial = r | jnp.left_shift(jnp.int32(1), 30 - i)
        cnt = jnp.sum((key < trial).astype(jnp.int32))
        return jnp.where(cnt < k, trial, r)

    vkey = lax.fori_loop(0, 31, sel_bit, jnp.int32(0))
    less = key < vkey
    cnt_less = jnp.sum(less.astype(jnp.int32))
    need_eq = k - cnt_less
    vloss = lax.bitcast_convert_type(vkey, jnp.float32)
    eq = key == vkey
    idx = (lax.broadcasted_iota(jnp.int32, (128, 128), 0) * 128
           + lax.broadcasted_iota(jnp.int32, (128, 128), 1))

    def sel_idx_bit(i, r):
        trial = r | jnp.left_shift(jnp.int32(1), 13 - i)
        cnt = jnp.sum((eq & (idx < trial)).astype(jnp.int32))
        return jnp.where(cnt < need_eq, trial, r)

    tidx = lax.fori_loop(0, 14, sel_idx_bit, jnp.int32(0))
    in_upd = less | (eq & (idx <= tidx))
    loss_clean = (jnp.sum(jnp.where(less, loss, 0.0))
                  + need_eq.astype(jnp.float32) * vloss) * inv_n
    mask_u1 = (idx >= 1) & (~in_upd)
    loss_dc = jnp.sum(jnp.where(mask_u1 & fc, ldc, 0.0)) * inv_n
    loss1 = jnp.sum(jnp.where(mask_u1 & (~fc), loss, 0.0)) * inv_n
    inter = jnp.sum(kl_ref[...]) * inv_n
    total = loss_clean + loss_dc + _DECAY_W * loss1 + _CO_LAMBDA * inter
    out_ref[...] = jnp.reshape(total, (1, 1))


def kernel(y_1, y_2, t, epoch):
    ep = jnp.asarray(epoch)
    rr_base = (1.0 - (0.5 / _EPOCHS) * ep).astype(jnp.float32).reshape(1, 1)
    thresh = (1.0 - (1.0 - min(0.5, 1.0 / _B)) * ep / _EPOCHS) \
        .astype(jnp.float32).reshape(1, 1)
    expnt = (0.5 - 0.5 * ep / _EPOCHS).astype(jnp.float32).reshape(1, 1)
    t2 = t.astype(jnp.int32).reshape(_B, 1)

    scalar_spec = pl.BlockSpec((1, 1), lambda i: (0, 0))
    row_spec = pl.BlockSpec((_RB, _CP), lambda i: (i, 0))
    col_spec = pl.BlockSpec((_RB, 1), lambda i: (i, 0))
    lp, ldc, fc, kl = pl.pallas_call(
        _rows_kernel,
        grid=(_B // _RB,),
        in_specs=[scalar_spec, scalar_spec, row_spec, row_spec, col_spec],
        out_specs=[col_spec] * 4,
        out_shape=[jax.ShapeDtypeStruct((_B, 1), jnp.float32)] * 4,
        compiler_params=pltpu.CompilerParams(
            dimension_semantics=("parallel",)),
    )(thresh, expnt, y_1, y_2, t2)

    out = pl.pallas_call(
        _select_kernel,
        in_specs=[pl.BlockSpec((1, 1), lambda: (0, 0))]
        + [pl.BlockSpec((128, 128), lambda: (0, 0))] * 4,
        out_specs=pl.BlockSpec((1, 1), lambda: (0, 0)),
        out_shape=jax.ShapeDtypeStruct((1, 1), jnp.float32),
    )(rr_base, lp.reshape(128, 128), ldc.reshape(128, 128),
      fc.reshape(128, 128), kl.reshape(128, 128))
    return out.reshape(())


# speedup vs baseline: 1.4367x; 1.1852x over previous
import jax
import jax.numpy as jnp
from jax import lax
from jax.experimental import pallas as pl
from jax.experimental.pallas import tpu as pltpu

_B = 16384
_C = 1000
_CP = 1024
_RB = 512
_STEPS = _B // _RB
_SUBT = _RB // 128
_EPOCHS = 100
_DECAY_W = 1.0
_CO_LAMBDA = 0.1
_FMIN = -3.0e38


def _fused_kernel(th_ref, ex_ref, rrb_ref, y1_ref, y2_ref, t_ref, out_ref,
                  lp_s, ldc_s, fc_s, kl_s):
    i = pl.program_id(0)
    thresh = th_ref[0, 0]
    expnt = ex_ref[0, 0]
    col = lax.broadcasted_iota(jnp.int32, (_RB, _CP), 1)
    valid = col < _C
    y1 = jnp.where(valid, y1_ref[...], _FMIN)
    y2 = jnp.where(valid, y2_ref[...], _FMIN)
    m1 = jnp.max(y1, axis=1, keepdims=True)
    m2 = jnp.max(y2, axis=1, keepdims=True)
    ex1 = jnp.exp(y1 - m1)
    ex2 = jnp.exp(y2 - m2)
    e1 = jnp.sum(ex1, axis=1, keepdims=True)
    e2 = jnp.sum(ex2, axis=1, keepdims=True)
    log_e1 = jnp.log(e1)
    log_e2 = jnp.log(e2)
    lse2 = m2 + log_e2
    am1 = y1 == m1
    tm = col == t_ref[...]
    yt1 = jnp.sum(jnp.where(tm, y1, 0.0), axis=1, keepdims=True)
    yt2 = jnp.sum(jnp.where(tm, y2, 0.0), axis=1, keepdims=True)
    ydc2 = jnp.sum(jnp.where(am1, y2, 0.0), axis=1, keepdims=True)
    lp = ((m1 + log_e1) - yt1) + (lse2 - yt2)
    q1 = ex1 * (1.0 / e1)
    q2 = ex2 * (1.0 / e2)
    kl = jnp.sum((q1 - q2) * (y1 - y2), axis=1, keepdims=True)
    pp = (1.0 / e1) * (1.0 / e2)
    fc = jnp.where((yt1 != m1) & (ydc2 == m2) & (pp > thresh), 1.0, 0.0)
    aw = jnp.exp(expnt * jnp.log(pp))
    ldc = aw * (log_e1 + (lse2 - ydc2))

    r0 = i * _SUBT
    lp_s[pl.ds(r0, _SUBT), :] = jnp.reshape(lp, (_SUBT, 128))
    ldc_s[pl.ds(r0, _SUBT), :] = jnp.reshape(ldc, (_SUBT, 128))
    fc_s[pl.ds(r0, _SUBT), :] = jnp.reshape(fc, (_SUBT, 128))
    kl_s[pl.ds(r0, _SUBT), :] = jnp.reshape(kl, (_SUBT, 128))

    @pl.when(i == _STEPS - 1)
    def _select():
        loss = lp_s[...]
        fcb = fc_s[...] > 0.5
        inv_n = jnp.float32(1.0 / _B)
        mean_v = jnp.sum(loss) * inv_n
        cnt_small = jnp.sum((loss < mean_v).astype(jnp.float32))
        rr = jnp.maximum(rrb_ref[0, 0], cnt_small * inv_n)
        k = jnp.floor(rr * _B).astype(jnp.int32)
        key = lax.bitcast_convert_type(loss, jnp.int32)

        def sel_bit(b, r):
            trial = r | jnp.left_shift(jnp.int32(1), 30 - b)
            cnt = jnp.sum((key < trial).astype(jnp.int32))
            return jnp.where(cnt < k, trial, r)

        vkey = lax.fori_loop(0, 31, sel_bit, jnp.int32(0))
        less = key < vkey
        cnt_less = jnp.sum(less.astype(jnp.int32))
        need_eq = k - cnt_less
        vloss = lax.bitcast_convert_type(vkey, jnp.float32)
        eq = key == vkey
        idx = (lax.broadcasted_iota(jnp.int32, (128, 128), 0) * 128
               + lax.broadcasted_iota(jnp.int32, (128, 128), 1))

        def sel_idx_bit(b, r):
            trial = r | jnp.left_shift(jnp.int32(1), 13 - b)
            cnt = jnp.sum((eq & (idx < trial)).astype(jnp.int32))
            return jnp.where(cnt < need_eq, trial, r)

        tidx = lax.fori_loop(0, 14, sel_idx_bit, jnp.int32(0))
        in_upd = less | (eq & (idx <= tidx))
        loss_clean = (jnp.sum(jnp.where(less, loss, 0.0))
                      + need_eq.astype(jnp.float32) * vloss) * inv_n
        mask_u1 = (idx >= 1) & (~in_upd)
        loss_dc = jnp.sum(jnp.where(mask_u1 & fcb, ldc_s[...], 0.0)) * inv_n
        loss1 = jnp.sum(jnp.where(mask_u1 & (~fcb), loss, 0.0)) * inv_n
        inter = jnp.sum(kl_s[...]) * inv_n
        total = (loss_clean + loss_dc + _DECAY_W * loss1
                 + _CO_LAMBDA * inter)
        out_ref[...] = jnp.reshape(total, (1, 1))


def kernel(y_1, y_2, t, epoch):
    ep = jnp.asarray(epoch)
    rr_base = (1.0 - (0.5 / _EPOCHS) * ep).astype(jnp.float32).reshape(1, 1)
    thresh = (1.0 - (1.0 - min(0.5, 1.0 / _B)) * ep / _EPOCHS) \
        .astype(jnp.float32).reshape(1, 1)
    expnt = (0.5 - 0.5 * ep / _EPOCHS).astype(jnp.float32).reshape(1, 1)
    t2 = t.astype(jnp.int32).reshape(_B, 1)

    scalar_spec = pl.BlockSpec((1, 1), lambda i: (0, 0))
    out = pl.pallas_call(
        _fused_kernel,
        grid=(_STEPS,),
        in_specs=[scalar_spec, scalar_spec, scalar_spec,
                  pl.BlockSpec((_RB, _CP), lambda i: (i, 0)),
                  pl.BlockSpec((_RB, _CP), lambda i: (i, 0)),
                  pl.BlockSpec((_RB, 1), lambda i: (i, 0))],
        out_specs=pl.BlockSpec((1, 1), lambda i: (0, 0)),
        out_shape=jax.ShapeDtypeStruct((1, 1), jnp.float32),
        scratch_shapes=[pltpu.VMEM((128, 128), jnp.float32)] * 4,
        compiler_params=pltpu.CompilerParams(
            dimension_semantics=("arbitrary",)),
    )(thresh, expnt, rr_base, y_1, y_2, t2)
    return out.reshape(())
